# baseline (speedup 1.0000x reference)
import dataclasses

import jax
import jax.numpy as jnp
from jax import lax
from jax.experimental import pallas as pl
from jax.experimental.pallas import tpu as pltpu
from jax.experimental.pallas import tpu_sc as plsc

_N_CORES = 2
_N_SUBCORES = 16
_N_LANES = 16
_N_WORKERS = _N_CORES * _N_SUBCORES

_BATCH = 16384
_D = 32
_BPW = _BATCH // _N_WORKERS
_GROUPS = _BPW // _N_LANES


def _mfpt_body(user_hbm, item_hbm, uf_hbm, if_hbm, ub_hbm, ib_hbm, out_hbm,
               uidx_v, iidx_v, ushift_v, ishift_v, urows_v, irows_v,
               ubrow_v, ibrow_v, out_v, sem):
    wid = lax.axis_index("s") * _N_CORES + lax.axis_index("c")
    base = wid * _BPW

    pltpu.sync_copy(user_hbm.at[pl.ds(base, _BPW)], uidx_v)
    pltpu.sync_copy(item_hbm.at[pl.ds(base, _BPW)], iidx_v)

    @pl.loop(0, _BPW, step=_N_LANES)
    def _(i):
        sl = pl.ds(i, _N_LANES)
        ushift_v[sl] = lax.shift_right_logical(uidx_v[sl], 4)
        ishift_v[sl] = lax.shift_right_logical(iidx_v[sl], 4)

    c1 = pltpu.async_copy(uf_hbm.at[uidx_v], urows_v, sem)
    c2 = pltpu.async_copy(if_hbm.at[iidx_v], irows_v, sem)
    c3 = pltpu.async_copy(ub_hbm.at[ushift_v], ubrow_v, sem)
    c4 = pltpu.async_copy(ib_hbm.at[ishift_v], ibrow_v, sem)
    c1.wait()
    c2.wait()
    c3.wait()
    c4.wait()

    lanes = lax.iota(jnp.int32, _N_LANES)

    @pl.loop(0, _GROUPS)
    def _(g):
        r0 = g * _N_LANES
        rows = lanes + r0
        ucol = lax.bitwise_and(uidx_v[pl.ds(r0, _N_LANES)], 15)
        icol = lax.bitwise_and(iidx_v[pl.ds(r0, _N_LANES)], 15)
        acc = plsc.load_gather(ubrow_v, [rows, ucol])
        acc = acc + plsc.load_gather(ibrow_v, [rows, icol])
        for j in range(_D):
            cols = lax.bitwise_and(lanes + j, _D - 1)
            au = plsc.load_gather(urows_v, [rows, cols])
            av = plsc.load_gather(irows_v, [rows, cols])
            acc = acc + au * av
        out_v[pl.ds(r0, _N_LANES)] = acc

    pltpu.sync_copy(out_v, out_hbm.at[pl.ds(base, _BPW)])


def kernel(user, item, user_factor, item_factor, users_biases, items_biases):
    user = user.astype(jnp.int32)
    item = item.astype(jnp.int32)
    n_users = users_biases.shape[0]
    n_items = items_biases.shape[0]
    ub_wide = users_biases.reshape(n_users // _N_LANES, _N_LANES)
    ib_wide = items_biases.reshape(n_items // _N_LANES, _N_LANES)
    mesh = plsc.VectorSubcoreMesh(
        core_axis_name="c", subcore_axis_name="s",
        num_cores=_N_CORES, num_subcores=_N_SUBCORES)
    cp = pltpu.CompilerParams()
    if "needs_layout_passes" in pltpu.CompilerParams.__dataclass_fields__:
        cp = dataclasses.replace(cp, needs_layout_passes=False)
    if "use_tc_tiling_on_sc" in pltpu.CompilerParams.__dataclass_fields__:
        cp = dataclasses.replace(cp, use_tc_tiling_on_sc=False)
    run = pl.kernel(
        _mfpt_body,
        compiler_params=cp,
        out_type=jax.ShapeDtypeStruct((_BATCH,), jnp.float32),
        mesh=mesh,
        scratch_types=[
            pltpu.VMEM((_BPW,), jnp.int32),
            pltpu.VMEM((_BPW,), jnp.int32),
            pltpu.VMEM((_BPW,), jnp.int32),
            pltpu.VMEM((_BPW,), jnp.int32),
            pltpu.VMEM((_BPW, _D), jnp.float32),
            pltpu.VMEM((_BPW, _D), jnp.float32),
            pltpu.VMEM((_BPW, _N_LANES), jnp.float32),
            pltpu.VMEM((_BPW, _N_LANES), jnp.float32),
            pltpu.VMEM((_BPW,), jnp.float32),
            pltpu.SemaphoreType.DMA,
        ],
    )
    return run(user, item, user_factor, item_factor, ub_wide, ib_wide)


# speedup vs baseline: 3.0161x; 3.0161x over previous
import dataclasses

import jax
import jax.numpy as jnp
from jax import lax
from jax.experimental import pallas as pl
from jax.experimental.pallas import tpu as pltpu
from jax.experimental.pallas import tpu_sc as plsc

_N_CORES = 2
_N_SUBCORES = 16
_N_LANES = 16
_N_WORKERS = _N_CORES * _N_SUBCORES

_BATCH = 16384
_D = 32
_BPW = _BATCH // _N_WORKERS
_NBUF = 8


def _fire(uft, ift, ub, ib, uslab, islab, ubg, ibg, sems, k, ju, ji):
    cu = pl.multiple_of(lax.shift_right_logical(ju, 7) * 128, 128)
    ci = pl.multiple_of(lax.shift_right_logical(ji, 7) * 128, 128)
    pltpu.async_copy(uft.at[:, pl.ds(cu, 128)], uslab.at[k], sems[k])
    pltpu.async_copy(ift.at[:, pl.ds(ci, 128)], islab.at[k], sems[k])
    pltpu.async_copy(ub.at[pl.ds(cu, 128)], ubg.at[k], sems[k])
    pltpu.async_copy(ib.at[pl.ds(ci, 128)], ibg.at[k], sems[k])


def _drain(uft, ub, uslab, islab, ubg, ibg, sems, k):
    pltpu.make_async_copy(uft.at[:, pl.ds(0, 128)], uslab.at[k],
                          sems[k]).wait()
    pltpu.make_async_copy(uft.at[:, pl.ds(0, 128)], islab.at[k],
                          sems[k]).wait()
    pltpu.make_async_copy(ub.at[pl.ds(0, 128)], ubg.at[k],
                          sems[k]).wait()
    pltpu.make_async_copy(ub.at[pl.ds(0, 128)], ibg.at[k],
                          sems[k]).wait()


def _mfpt_body(user_hbm, item_hbm, uft_hbm, ift_hbm, ub_hbm, ib_hbm, out_hbm,
               uidx_v, iidx_v, uslab_v, islab_v, ubg_v, ibg_v, res_v, out_v,
               *sems):
    wid = lax.axis_index("s") * _N_CORES + lax.axis_index("c")
    base = wid * _BPW

    pltpu.sync_copy(user_hbm.at[pl.ds(base, _BPW)], uidx_v)
    pltpu.sync_copy(item_hbm.at[pl.ds(base, _BPW)], iidx_v)

    lanes = lax.iota(jnp.int32, _N_LANES)
    c_lo = lanes
    c_hi = lanes + _N_LANES

    def compute(n, k, ju, ji):
        lu = lax.broadcast(lax.bitwise_and(ju, 127), (_N_LANES,))
        li = lax.broadcast(lax.bitwise_and(ji, 127), (_N_LANES,))
        u1 = plsc.load_gather(uslab_v.at[k], [c_lo, lu])
        u2 = plsc.load_gather(uslab_v.at[k], [c_hi, lu])
        v1 = plsc.load_gather(islab_v.at[k], [c_lo, li])
        v2 = plsc.load_gather(islab_v.at[k], [c_hi, li])
        p = u1 * v1 + u2 * v2
        s = jnp.sum(p)
        bu = plsc.load_gather(ubg_v, [lanes * 0 + k, lu])
        bi = plsc.load_gather(ibg_v, [lanes * 0 + k, li])
        tot = s + bu[0] + bi[0]
        m = lanes == lax.bitwise_and(n, 15)
        res_v[...] = lax.select(m, lax.broadcast(tot, (_N_LANES,)), res_v[...])

    vju0 = uidx_v[pl.ds(0, _N_LANES)]
    vji0 = iidx_v[pl.ds(0, _N_LANES)]
    for k in range(_NBUF):
        _fire(uft_hbm, ift_hbm, ub_hbm, ib_hbm,
              uslab_v, islab_v, ubg_v, ibg_v, sems, k, vju0[k], vji0[k])

    @pl.loop(0, _BPW - _NBUF, step=_NBUF)
    def _(i8):
        vju = uidx_v[pl.ds(i8, 2 * _NBUF)]
        vji = iidx_v[pl.ds(i8, 2 * _NBUF)]
        for k in range(_NBUF):
            _drain(uft_hbm, ub_hbm, uslab_v, islab_v, ubg_v, ibg_v, sems, k)
            compute(i8 + k, k, vju[k], vji[k])
            _fire(uft_hbm, ift_hbm, ub_hbm, ib_hbm,
                  uslab_v, islab_v, ubg_v, ibg_v, sems, k,
                  vju[k + _NBUF], vji[k + _NBUF])
        out_v[pl.ds(lax.bitwise_and(i8, -16), _N_LANES)] = res_v[...]

    vjuN = uidx_v[pl.ds(_BPW - _N_LANES, _N_LANES)]
    vjiN = iidx_v[pl.ds(_BPW - _N_LANES, _N_LANES)]
    for k in range(_NBUF):
        _drain(uft_hbm, ub_hbm, uslab_v, islab_v, ubg_v, ibg_v, sems, k)
        compute(_BPW - _NBUF + k, k, vjuN[k + _NBUF], vjiN[k + _NBUF])
    out_v[pl.ds(_BPW - _N_LANES, _N_LANES)] = res_v[...]

    pltpu.sync_copy(out_v, out_hbm.at[pl.ds(base, _BPW)])


def kernel(user, item, user_factor, item_factor, users_biases, items_biases):
    user = user.astype(jnp.int32)
    item = item.astype(jnp.int32)
    uft = user_factor.T
    ift = item_factor.T
    ub = users_biases.reshape(-1)
    ib = items_biases.reshape(-1)
    mesh = plsc.VectorSubcoreMesh(
        core_axis_name="c", subcore_axis_name="s",
        num_cores=_N_CORES, num_subcores=_N_SUBCORES)
    cp = pltpu.CompilerParams()
    if "needs_layout_passes" in pltpu.CompilerParams.__dataclass_fields__:
        cp = dataclasses.replace(cp, needs_layout_passes=False)
    if "use_tc_tiling_on_sc" in pltpu.CompilerParams.__dataclass_fields__:
        cp = dataclasses.replace(cp, use_tc_tiling_on_sc=True)
    run = pl.kernel(
        _mfpt_body,
        compiler_params=cp,
        out_type=jax.ShapeDtypeStruct((_BATCH,), jnp.float32),
        mesh=mesh,
        scratch_types=[
            pltpu.VMEM((_BPW,), jnp.int32),
            pltpu.VMEM((_BPW,), jnp.int32),
            pltpu.VMEM((_NBUF, _D, 128), jnp.float32),
            pltpu.VMEM((_NBUF, _D, 128), jnp.float32),
            pltpu.VMEM((_NBUF, 128), jnp.float32),
            pltpu.VMEM((_NBUF, 128), jnp.float32),
            pltpu.VMEM((_N_LANES,), jnp.float32),
            pltpu.VMEM((_BPW,), jnp.float32),
        ] + [pltpu.SemaphoreType.DMA] * _NBUF,
    )
    return run(user, item, uft, ift, ub, ib)
